# baseline (speedup 1.0000x reference)
import functools

import jax
import jax.numpy as jnp
from jax import lax
from jax.experimental import pallas as pl
from jax.experimental.pallas import tpu as pltpu
from jax.experimental.pallas import tpu_sc as plsc

F32 = jnp.float32

_NC = 2
_NS = 16
_NT = _NC * _NS
_IW = 125
_CH = 8


def _t(x):
    return jnp.transpose(x, (1, 0))


def _dot(a, b):
    return jnp.dot(a, b, preferred_element_type=F32)


def _ln_s_t(s, gamma_c, beta_c):
    mu = jnp.mean(s, axis=0, keepdims=True)
    var = jnp.mean(jnp.square(s - mu), axis=0, keepdims=True)
    return (s - mu) / jnp.sqrt(var + 1e-5) * gamma_c + beta_c


def _ln_v_t(vx, vy, vz):
    n2 = jnp.maximum(vx * vx + vy * vy + vz * vz, 1e-8)
    r = 1.0 / jnp.sqrt(jnp.mean(n2, axis=0, keepdims=True))
    return vx * r, vy * r, vz * r


def _gvp_t(s, vx, vy, vz, wht, wswt, wsbt, wvt, do_relu, do_gate):
    vhx, vhy, vhz = _dot(wht, vx), _dot(wht, vy), _dot(wht, vz)
    vn = jnp.sqrt(jnp.maximum(vhx * vhx + vhy * vhy + vhz * vhz, 1e-8))
    so = _dot(wswt, jnp.concatenate([s, vn], axis=0)) + wsbt
    if do_relu:
        so = jnp.maximum(so, 0.0)
    if wvt is None:
        return so, None, None, None
    vox, voy, voz = _dot(wvt, vhx), _dot(wvt, vhy), _dot(wvt, vhz)
    if do_gate:
        g = jax.nn.sigmoid(
            jnp.sqrt(jnp.maximum(vox * vox + voy * voy + voz * voz, 1e-8)))
        vox, voy, voz = vox * g, voy * g, voz * g
    return so, vox, voy, voz


def _pack16_t(s, vx, vy, vz, count_lane):
    b = s.shape[1]
    fill = jnp.ones((1, b), F32) if count_lane else jnp.zeros((1, b), F32)
    z = jnp.zeros((5, b), F32)
    return jnp.concatenate([s, vx, vy, vz, fill, z], axis=0)


def _unpack128(x128):
    xt = _t(x128)
    return jnp.concatenate([xt[16 * j:16 * j + 16] for j in range(8)],
                           axis=1)


def _pack128(m16):
    sub = m16.shape[1] // 8
    y = jnp.concatenate([m16[:, sub * j:sub * j + sub] for j in range(8)],
                        axis=0)
    return _t(y)


def _col(w):
    return w.reshape(-1, 1)


def _full(shape):
    nd = len(shape)
    return pl.BlockSpec(shape, lambda i: (0,) * nd)


def _rows(bs, width, block_off=0):
    return pl.BlockSpec((bs, width), lambda i, o=block_off: (i + o, 0))


def _cols(height, bs, block_row=0):
    return pl.BlockSpec((height, bs), lambda i, r=block_row: (r, i))


def _node_embed_call(ns_t, nv_t, params):
    n = ns_t.shape[1]
    ln, g = params['Wv_ln'], params['Wv_gvp']

    def body(s_ref, v_ref, lg_ref, lb_ref, wh_ref, wsw_ref, wsb_ref, wv_ref,
             out_ref):
        s = _ln_s_t(s_ref[...], lg_ref[...], lb_ref[...])
        v = v_ref[...]
        vx, vy, vz = _ln_v_t(v[0:2], v[2:4], v[4:6])
        so, ox, oy, oz = _gvp_t(s, vx, vy, vz, wh_ref[...], wsw_ref[...],
                                wsb_ref[...], wv_ref[...], False, False)
        out_ref[...] = _t(_pack16_t(so, ox, oy, oz, False))

    return pl.pallas_call(
        body,
        grid=(1,),
        in_specs=[_full((4, n)), _full((6, n)), _full((4, 1)), _full((4, 1)),
                  _full((2, 2)), _full((4, 6)), _full((4, 1)), _full((2, 2))],
        out_specs=_full((n, 16)),
        out_shape=jax.ShapeDtypeStruct((n, 16), F32),
    )(ns_t, nv_t, _col(ln['gamma']), _col(ln['beta']), g['wh'].T,
      g['ws_w'].T, _col(g['ws_b']), g['wv'].T)


def _edge_embed_call(es_t, ev_t, params):
    e = es_t.shape[1]
    be = 16000
    ln, g = params['We_ln'], params['We_gvp']
    w2t = jnp.concatenate(
        [params['layers'][l]['msg'][0]['ws_w'][4:36].T for l in range(2)], 0)
    b2t = jnp.concatenate(
        [_col(params['layers'][l]['msg'][0]['ws_b']) for l in range(2)], 0)

    def body(es_ref, ev_ref, lg_ref, lb_ref, wh_ref, wsw_ref, wsb_ref, wv_ref,
             w2_ref, b2_ref, p_ref, ev_out_ref):
        es = _ln_s_t(es_ref[...], lg_ref[...], lb_ref[...])
        ev = ev_ref[...]
        whs = wh_ref[0, 0]
        vh = ev * whs
        vn = jnp.sqrt(jnp.maximum(
            jnp.sum(vh * vh, axis=0, keepdims=True), 1e-8))
        eso = _dot(wsw_ref[...], jnp.concatenate([es, vn], axis=0)) \
            + wsb_ref[...]
        p_ref[...] = _dot(w2_ref[...], eso) + b2_ref[...]
        ev_out_ref[...] = vh * wv_ref[0, 0]

    return pl.pallas_call(
        body,
        grid=(e // be,),
        in_specs=[_cols(32, be), _cols(3, be), _full((32, 1)), _full((32, 1)),
                  _full((1, 1)), _full((32, 33)), _full((32, 1)),
                  _full((1, 1)), _full((8, 32)), _full((8, 1))],
        out_specs=[_cols(8, be), _cols(3, be)],
        out_shape=[jax.ShapeDtypeStruct((8, e), F32),
                   jax.ShapeDtypeStruct((3, e), F32)],
    )(es_t, ev_t, _col(ln['gamma']), _col(ln['beta']), g['wh'],
      g['ws_w'].T, _col(g['ws_b']), g['wv'], w2t, b2t)


def _msg_call(gout, eproj8, evemb, lp, layer_idx):
    e = eproj8.shape[1]
    be = 16000
    m0, m1, m2 = lp['msg']

    sub = be // 8

    def _unpack(x128):
        xt = _t(x128)
        return jnp.concatenate([xt[16 * j:16 * j + 16] for j in range(8)],
                               axis=1)

    def body(gs_ref, gd_ref, ep_ref, ev_ref, wh0_ref, wsd_ref, wvn_ref,
             wv0_ref, wh1_ref, wsw1_ref, wsb1_ref, wv1_ref,
             wh2_ref, wsw2_ref, wsb2_ref, wv2_ref, out_ref):
        gs = _unpack(gs_ref[...])
        gd = _unpack(gd_ref[...])
        ev = ev_ref[...]
        wh0 = wh0_ref[...]
        mvx = jnp.concatenate([gs[4:6], ev[0:1], gd[4:6]], axis=0)
        mvy = jnp.concatenate([gs[6:8], ev[1:2], gd[6:8]], axis=0)
        mvz = jnp.concatenate([gs[8:10], ev[2:3], gd[8:10]], axis=0)
        vhx, vhy, vhz = _dot(wh0, mvx), _dot(wh0, mvy), _dot(wh0, mvz)
        vn = jnp.sqrt(jnp.maximum(vhx * vhx + vhy * vhy + vhz * vhz, 1e-8))
        wsd = wsd_ref[...]
        ep = ep_ref[...][4 * layer_idx:4 * layer_idx + 4]
        s1 = _dot(wsd, jnp.concatenate([gs[0:4], gd[0:4]], axis=0)) \
            + ep + _dot(wvn_ref[...], vn)
        s1 = jnp.maximum(s1, 0.0)
        wv0 = wv0_ref[...]
        vox, voy, voz = _dot(wv0, vhx), _dot(wv0, vhy), _dot(wv0, vhz)
        g = jax.nn.sigmoid(
            jnp.sqrt(jnp.maximum(vox * vox + voy * voy + voz * voz, 1e-8)))
        vox, voy, voz = vox * g, voy * g, voz * g
        s2, v2x, v2y, v2z = _gvp_t(s1, vox, voy, voz, wh1_ref[...],
                                   wsw1_ref[...], wsb1_ref[...], wv1_ref[...],
                                   True, True)
        s3, v3x, v3y, v3z = _gvp_t(s2, v2x, v2y, v2z, wh2_ref[...],
                                   wsw2_ref[...], wsb2_ref[...], wv2_ref[...],
                                   False, False)
        m16 = _pack16_t(s3, v3x, v3y, v3z, True)
        y = jnp.concatenate(
            [m16[:, sub * j:sub * j + sub] for j in range(8)], axis=0)
        out_ref[...] = _t(y)

    nblk = e // be
    wsd = jnp.concatenate([m0['ws_w'][0:4].T, m0['ws_w'][36:40].T], axis=1)
    g128 = gout.reshape(-1, 128)
    return pl.pallas_call(
        body,
        grid=(nblk,),
        in_specs=[_rows(be // 8, 128), _rows(be // 8, 128, block_off=nblk),
                  _cols(8, be), _cols(3, be),
                  _full((5, 5)), _full((4, 8)), _full((4, 5)), _full((2, 5)),
                  _full((2, 2)), _full((4, 6)), _full((4, 1)), _full((2, 2)),
                  _full((2, 2)), _full((4, 6)), _full((4, 1)), _full((2, 2))],
        out_specs=_rows(be // 8, 128),
        out_shape=jax.ShapeDtypeStruct((e // 8, 128), F32),
    )(g128, g128, eproj8, evemb,
      m0['wh'].T, wsd, m0['ws_w'][40:45].T, m0['wv'].T,
      m1['wh'].T, m1['ws_w'].T, _col(m1['ws_b']), m1['wv'].T,
      m2['wh'].T, m2['ws_w'].T, _col(m2['ws_b']), m2['wv'].T)


def _update_common(t_ref, p0_ref, p1_ref, lp_refs):
    (g0_ref, b0_ref, fwh0_ref, fwsw0_ref, fwsb0_ref, fwv0_ref,
     fwh1_ref, fwsw1_ref, fwsb1_ref, fwv1_ref, g1_ref, b1_ref) = lp_refs
    t = _unpack128(t_ref[...])
    agg = _unpack128(p0_ref[0]) + _unpack128(p1_ref[0])
    inv = 1.0 / jnp.maximum(agg[10:11], 1.0)
    s = t[0:4] + agg[0:4] * inv
    vx = t[4:6] + agg[4:6] * inv
    vy = t[6:8] + agg[6:8] * inv
    vz = t[8:10] + agg[8:10] * inv
    s = _ln_s_t(s, g0_ref[...], b0_ref[...])
    vx, vy, vz = _ln_v_t(vx, vy, vz)
    hs, hx, hy, hz = _gvp_t(s, vx, vy, vz, fwh0_ref[...], fwsw0_ref[...],
                            fwsb0_ref[...], fwv0_ref[...], True, True)
    hs, hx, hy, hz = _gvp_t(hs, hx, hy, hz, fwh1_ref[...], fwsw1_ref[...],
                            fwsb1_ref[...], fwv1_ref[...], False, False)
    s = _ln_s_t(s + hs, g1_ref[...], b1_ref[...])
    vx, vy, vz = _ln_v_t(vx + hx, vy + hy, vz + hz)
    return s, vx, vy, vz


_UPD_WSPECS = [
    _full((4, 1)), _full((4, 1)),
    _full((4, 2)), _full((16, 8)), _full((16, 1)), _full((4, 4)),
    _full((4, 4)), _full((4, 20)), _full((4, 1)), _full((2, 4)),
    _full((4, 1)), _full((4, 1)),
]


def _upd_weights(lp):
    f0, f1 = lp['ff']
    return (_col(lp['norm0']['gamma']), _col(lp['norm0']['beta']),
            f0['wh'].T, f0['ws_w'].T, _col(f0['ws_b']), f0['wv'].T,
            f1['wh'].T, f1['ws_w'].T, _col(f1['ws_b']), f1['wv'].T,
            _col(lp['norm1']['gamma']), _col(lp['norm1']['beta']))


def _update_call(table, part, lp):
    n = table.shape[0]
    nblk = 1
    br = n // 8

    def body(t_ref, p0_ref, p1_ref, *rest):
        lp_refs, out_ref = rest[:-1], rest[-1]
        s, vx, vy, vz = _update_common(t_ref, p0_ref, p1_ref, lp_refs)
        out_ref[...] = _pack128(_pack16_t(s, vx, vy, vz, False))

    t128 = table.reshape(n // 8, 128)
    p128 = part.reshape(2, n // 8, 128)
    half = lambda h: pl.BlockSpec((1, n // 8, 128), lambda i, h=h: (h, 0, 0))
    return pl.pallas_call(
        body,
        grid=(nblk,),
        in_specs=[_rows(br, 128), half(0), half(1)]
        + _UPD_WSPECS,
        out_specs=_rows(br, 128),
        out_shape=jax.ShapeDtypeStruct((n // 8, 128), F32),
    )(t128, p128, p128, *_upd_weights(lp)).reshape(n, 16)


def _update_head_call(table, part, lp, params):
    n = table.shape[0]
    nblk = 1
    br = n // 8
    og = params['Wout_gvp']

    def body(t_ref, p0_ref, p1_ref, *rest):
        lp_refs = rest[:12]
        (og_ref, ob_ref, owh_ref, owsw_ref, owsb_ref, fw_ref, fb_ref,
         out_ref) = rest[12:]
        s, vx, vy, vz = _update_common(t_ref, p0_ref, p1_ref, lp_refs)
        s = _ln_s_t(s, og_ref[...], ob_ref[...])
        vx, vy, vz = _ln_v_t(vx, vy, vz)
        so, _, _, _ = _gvp_t(s, vx, vy, vz, owh_ref[...], owsw_ref[...],
                             owsb_ref[...], None, True, False)
        o1 = _dot(fw_ref[...], so) + fb_ref[...]
        y = jnp.concatenate(
            [o1[:, br * j:br * j + br] for j in range(8)], axis=0)
        out_ref[...] = _t(y)

    t128 = table.reshape(n // 8, 128)
    p128 = part.reshape(2, n // 8, 128)
    half = lambda h: pl.BlockSpec((1, n // 8, 128), lambda i, h=h: (h, 0, 0))
    return pl.pallas_call(
        body,
        grid=(nblk,),
        in_specs=[_rows(br, 128), half(0), half(1)]
        + _UPD_WSPECS
        + [_full((4, 1)), _full((4, 1)), _full((2, 2)), _full((4, 6)),
           _full((4, 1)), _full((1, 4)), _full((1, 1))],
        out_specs=_rows(br, 8),
        out_shape=jax.ShapeDtypeStruct((n // 8, 8), F32),
    )(t128, p128, p128, *_upd_weights(lp),
      _col(params['Wout_ln']['gamma']), _col(params['Wout_ln']['beta']),
      og['wh'].T, og['ws_w'].T, _col(og['ws_b']),
      params['final_w'].T, params['final_b'].reshape(1, 1))


def _gather_call(table, gidx):
    n = table.shape[0]
    r_total = gidx.shape[0]
    rows_total = r_total * _IW
    rpt = r_total // _NT
    n_chunks = rpt // _CH
    n_stage = n // 1000
    mesh = plsc.VectorSubcoreMesh(core_axis_name="c", subcore_axis_name="s")

    @functools.partial(
        pl.kernel,
        out_type=jax.ShapeDtypeStruct((rows_total, 16), F32),
        mesh=mesh,
        compiler_params=pltpu.CompilerParams(use_tc_tiling_on_sc=False),
        scratch_types=[
            pltpu.VMEM((_CH, _IW), jnp.int32),
            pltpu.VMEM((_CH * _IW, 16), F32),
            pltpu.VMEM((_CH * _IW, 16), F32),
            pltpu.VMEM_SHARED((n, 16), F32),
            pltpu.SemaphoreType.DMA,
            pltpu.SemaphoreType.DMA,
        ],
    )
    def k(table_hbm, idx_hbm, out_hbm, idx_v, rows_a, rows_b, tab_sh, gsem,
          osem):
        cid = lax.axis_index("c")
        sid = lax.axis_index("s")
        for z in range(-(-n_stage // _NS)):
            c = sid + z * _NS

            @pl.when(c < n_stage)
            def _():
                pltpu.sync_copy(table_hbm.at[pl.ds(c * 1000, 1000)],
                                tab_sh.at[pl.ds(c * 1000, 1000)])
        plsc.subcore_barrier()
        wid = sid * _NC + cid
        row0 = wid * rpt

        def do_chunk(ci, buf):
            r0 = row0 + ci * _CH
            pltpu.sync_copy(idx_hbm.at[pl.ds(r0, _CH)], idx_v)
            cps = [pltpu.async_copy(tab_sh.at[idx_v.at[j]],
                                    buf.at[pl.ds(j * _IW, _IW)], gsem)
                   for j in range(_CH)]
            for c in cps:
                c.wait()
            pltpu.async_copy(buf, out_hbm.at[pl.ds(r0 * _IW, _CH * _IW)],
                             osem)

        def drain(ci, buf):
            r0 = row0 + ci * _CH
            pltpu.make_async_copy(
                buf, out_hbm.at[pl.ds(r0 * _IW, _CH * _IW)], osem).wait()

        def pair(p, carry):
            c0 = p * 2

            @pl.when(p > 0)
            def _():
                drain(c0 - 2, rows_a)
            do_chunk(c0, rows_a)

            @pl.when(p > 0)
            def _():
                drain(c0 - 1, rows_b)
            do_chunk(c0 + 1, rows_b)
            return carry

        lax.fori_loop(0, n_chunks // 2, pair, 0)
        drain(n_chunks - 2, rows_a)
        drain(n_chunks - 1, rows_b)

    return k(table, gidx)


def _scatter_call(msgs, didx, zrows, n):
    r_total = didx.shape[0]
    rpt = r_total // _NT
    n_chunks = rpt // _CH
    zch = zrows.shape[0]
    n_z = n // zch
    mesh = plsc.VectorSubcoreMesh(core_axis_name="c", subcore_axis_name="s")

    @functools.partial(
        pl.kernel,
        out_type=jax.ShapeDtypeStruct((2 * n, 16), F32),
        mesh=mesh,
        compiler_params=pltpu.CompilerParams(use_tc_tiling_on_sc=False),
        scratch_types=[
            pltpu.VMEM((_CH, _IW), jnp.int32),
            pltpu.VMEM((_CH * _IW, 16), F32),
            pltpu.VMEM_SHARED((n, 16), F32),
            pltpu.SemaphoreType.DMA,
        ],
    )
    def k(msg_hbm, idx_hbm, z_hbm, out_hbm, idx_v, msg_v, agg_sh, sem):
        cid = lax.axis_index("c")
        sid = lax.axis_index("s")
        for z in range(-(-n_z // _NS)):
            c = sid + z * _NS

            @pl.when(c < n_z)
            def _():
                pltpu.sync_copy(z_hbm, agg_sh.at[pl.ds(c * zch, zch)])
        plsc.subcore_barrier()

        def chunk(ci, carry):
            r0 = (cid * _NS + sid) * rpt + ci * _CH
            ld0 = pltpu.async_copy(idx_hbm.at[pl.ds(r0, _CH)], idx_v, sem)
            ld1 = pltpu.async_copy(
                msg_hbm.at[pl.ds(r0 * _IW, _CH * _IW)], msg_v, sem)
            ld0.wait()
            ld1.wait()
            cps = [pltpu.async_copy(msg_v.at[pl.ds(j * _IW, _IW)],
                                    agg_sh.at[idx_v.at[j]], sem, add=True)
                   for j in range(_CH)]
            for c in cps:
                c.wait()
            return carry

        lax.fori_loop(0, n_chunks, chunk, 0)
        plsc.subcore_barrier()
        for z in range(-(-n_z // _NS)):
            c = sid + z * _NS

            @pl.when(c < n_z)
            def _():
                pltpu.sync_copy(agg_sh.at[pl.ds(c * zch, zch)],
                                out_hbm.at[pl.ds(cid * n + c * zch, zch)])

    return k(msgs, didx, zrows)


def kernel(node_s, node_v, edge_index, edge_s, edge_v, params):
    n = node_s.shape[0]
    e = edge_s.shape[0]
    ns_t = node_s.T
    nv_t = jnp.transpose(node_v, (2, 1, 0)).reshape(6, n)
    es_t = edge_s.T
    ev_t = jnp.transpose(edge_v, (2, 1, 0)).reshape(3, e)
    def _perm(a):
        return a.reshape(-1, 8, 2000).swapaxes(1, 2).reshape(a.shape)

    gidx = _perm(edge_index.reshape(-1)).reshape(2 * e // _IW, _IW)
    didx = gidx[e // _IW:]
    zrows = jnp.zeros((1000, 16), F32)

    table = _node_embed_call(ns_t, nv_t, params)
    eproj8, evemb = _edge_embed_call(es_t, ev_t, params)

    out = None
    for li, lp in enumerate(params['layers']):
        gout = _gather_call(table, gidx)
        msgs = _msg_call(gout, eproj8, evemb, lp, li)
        part = _scatter_call(msgs.reshape(e, 16), didx, zrows, n)
        if li == 0:
            table = _update_call(table, part, lp)
        else:
            out = _update_head_call(table, part, lp, params)
    return out.reshape(n)


# speedup vs baseline: 1.0628x; 1.0628x over previous
import functools

import jax
import jax.numpy as jnp
from jax import lax
from jax.experimental import pallas as pl
from jax.experimental.pallas import tpu as pltpu
from jax.experimental.pallas import tpu_sc as plsc

F32 = jnp.float32

_NC = 2
_NS = 16
_NT = _NC * _NS
_IW = 125
_CH = 8


def _t(x):
    return jnp.transpose(x, (1, 0))


def _dot(a, b):
    return jnp.dot(a, b, preferred_element_type=F32)


def _ln_s_t(s, gamma_c, beta_c):
    mu = jnp.mean(s, axis=0, keepdims=True)
    var = jnp.mean(jnp.square(s - mu), axis=0, keepdims=True)
    return (s - mu) / jnp.sqrt(var + 1e-5) * gamma_c + beta_c


def _ln_v_t(vx, vy, vz):
    n2 = jnp.maximum(vx * vx + vy * vy + vz * vz, 1e-8)
    r = 1.0 / jnp.sqrt(jnp.mean(n2, axis=0, keepdims=True))
    return vx * r, vy * r, vz * r


def _gvp_t(s, vx, vy, vz, wht, wswt, wsbt, wvt, do_relu, do_gate):
    vhx, vhy, vhz = _dot(wht, vx), _dot(wht, vy), _dot(wht, vz)
    vn = jnp.sqrt(jnp.maximum(vhx * vhx + vhy * vhy + vhz * vhz, 1e-8))
    so = _dot(wswt, jnp.concatenate([s, vn], axis=0)) + wsbt
    if do_relu:
        so = jnp.maximum(so, 0.0)
    if wvt is None:
        return so, None, None, None
    vox, voy, voz = _dot(wvt, vhx), _dot(wvt, vhy), _dot(wvt, vhz)
    if do_gate:
        g = jax.nn.sigmoid(
            jnp.sqrt(jnp.maximum(vox * vox + voy * voy + voz * voz, 1e-8)))
        vox, voy, voz = vox * g, voy * g, voz * g
    return so, vox, voy, voz


def _pack16_t(s, vx, vy, vz, count_lane):
    b = s.shape[1]
    fill = jnp.ones((1, b), F32) if count_lane else jnp.zeros((1, b), F32)
    z = jnp.zeros((5, b), F32)
    return jnp.concatenate([s, vx, vy, vz, fill, z], axis=0)


def _unpack128(x128):
    xt = _t(x128)
    return jnp.concatenate([xt[16 * j:16 * j + 16] for j in range(8)],
                           axis=1)


def _pack128(m16):
    sub = m16.shape[1] // 8
    y = jnp.concatenate([m16[:, sub * j:sub * j + sub] for j in range(8)],
                        axis=0)
    return _t(y)


def _col(w):
    return w.reshape(-1, 1)


def _full(shape):
    nd = len(shape)
    return pl.BlockSpec(shape, lambda i: (0,) * nd)


def _rows(bs, width, block_off=0):
    return pl.BlockSpec((bs, width), lambda i, o=block_off: (i + o, 0))


def _cols(height, bs, block_row=0):
    return pl.BlockSpec((height, bs), lambda i, r=block_row: (r, i))


def _node_embed_call(ns_t, nv_t, params):
    n = ns_t.shape[1]
    ln, g = params['Wv_ln'], params['Wv_gvp']

    def body(s_ref, v_ref, lg_ref, lb_ref, wh_ref, wsw_ref, wsb_ref, wv_ref,
             out_ref):
        s = _ln_s_t(s_ref[...], lg_ref[...], lb_ref[...])
        v = v_ref[...]
        vx, vy, vz = _ln_v_t(v[0:2], v[2:4], v[4:6])
        so, ox, oy, oz = _gvp_t(s, vx, vy, vz, wh_ref[...], wsw_ref[...],
                                wsb_ref[...], wv_ref[...], False, False)
        out_ref[...] = _t(_pack16_t(so, ox, oy, oz, False))

    return pl.pallas_call(
        body,
        grid=(1,),
        in_specs=[_full((4, n)), _full((6, n)), _full((4, 1)), _full((4, 1)),
                  _full((2, 2)), _full((4, 6)), _full((4, 1)), _full((2, 2))],
        out_specs=_full((n, 16)),
        out_shape=jax.ShapeDtypeStruct((n, 16), F32),
    )(ns_t, nv_t, _col(ln['gamma']), _col(ln['beta']), g['wh'].T,
      g['ws_w'].T, _col(g['ws_b']), g['wv'].T)


def _edge_embed_call(es_t, ev_t, params):
    e = es_t.shape[1]
    be = 16000
    ln, g = params['We_ln'], params['We_gvp']
    w2t = jnp.concatenate(
        [params['layers'][l]['msg'][0]['ws_w'][4:36].T for l in range(2)], 0)
    b2t = jnp.concatenate(
        [_col(params['layers'][l]['msg'][0]['ws_b']) for l in range(2)], 0)

    def body(es_ref, ev_ref, lg_ref, lb_ref, wh_ref, wsw_ref, wsb_ref, wv_ref,
             w2_ref, b2_ref, p_ref, ev_out_ref):
        es = _ln_s_t(es_ref[...], lg_ref[...], lb_ref[...])
        ev = ev_ref[...]
        whs = wh_ref[0, 0]
        vh = ev * whs
        vn = jnp.sqrt(jnp.maximum(
            jnp.sum(vh * vh, axis=0, keepdims=True), 1e-8))
        eso = _dot(wsw_ref[...], jnp.concatenate([es, vn], axis=0)) \
            + wsb_ref[...]
        p_ref[...] = _dot(w2_ref[...], eso) + b2_ref[...]
        ev_out_ref[...] = vh * wv_ref[0, 0]

    return pl.pallas_call(
        body,
        grid=(e // be,),
        in_specs=[_cols(32, be), _cols(3, be), _full((32, 1)), _full((32, 1)),
                  _full((1, 1)), _full((32, 33)), _full((32, 1)),
                  _full((1, 1)), _full((8, 32)), _full((8, 1))],
        out_specs=[_cols(8, be), _cols(3, be)],
        out_shape=[jax.ShapeDtypeStruct((8, e), F32),
                   jax.ShapeDtypeStruct((3, e), F32)],
    )(es_t, ev_t, _col(ln['gamma']), _col(ln['beta']), g['wh'],
      g['ws_w'].T, _col(g['ws_b']), g['wv'], w2t, b2t)


def _msg_call(gout, eproj8, evemb, lp, layer_idx):
    e = eproj8.shape[1]
    be = 16000
    m0, m1, m2 = lp['msg']

    sub = be // 8

    def _unpack(x128):
        xt = _t(x128)
        return jnp.concatenate([xt[16 * j:16 * j + 16] for j in range(8)],
                               axis=1)

    def body(gs_ref, gd_ref, ep_ref, ev_ref, wh0_ref, wsd_ref, wvn_ref,
             wv0_ref, wh1_ref, wsw1_ref, wsb1_ref, wv1_ref,
             wh2_ref, wsw2_ref, wsb2_ref, wv2_ref, out_ref):
        gs = _unpack(gs_ref[...])
        gd = _unpack(gd_ref[...])
        ev = ev_ref[...]
        wh0 = wh0_ref[...]
        mvx = jnp.concatenate([gs[4:6], ev[0:1], gd[4:6]], axis=0)
        mvy = jnp.concatenate([gs[6:8], ev[1:2], gd[6:8]], axis=0)
        mvz = jnp.concatenate([gs[8:10], ev[2:3], gd[8:10]], axis=0)
        vhx, vhy, vhz = _dot(wh0, mvx), _dot(wh0, mvy), _dot(wh0, mvz)
        vn = jnp.sqrt(jnp.maximum(vhx * vhx + vhy * vhy + vhz * vhz, 1e-8))
        wsd = wsd_ref[...]
        ep = ep_ref[...][4 * layer_idx:4 * layer_idx + 4]
        s1 = _dot(wsd, jnp.concatenate([gs[0:4], gd[0:4]], axis=0)) \
            + ep + _dot(wvn_ref[...], vn)
        s1 = jnp.maximum(s1, 0.0)
        wv0 = wv0_ref[...]
        vox, voy, voz = _dot(wv0, vhx), _dot(wv0, vhy), _dot(wv0, vhz)
        g = jax.nn.sigmoid(
            jnp.sqrt(jnp.maximum(vox * vox + voy * voy + voz * voz, 1e-8)))
        vox, voy, voz = vox * g, voy * g, voz * g
        s2, v2x, v2y, v2z = _gvp_t(s1, vox, voy, voz, wh1_ref[...],
                                   wsw1_ref[...], wsb1_ref[...], wv1_ref[...],
                                   True, True)
        s3, v3x, v3y, v3z = _gvp_t(s2, v2x, v2y, v2z, wh2_ref[...],
                                   wsw2_ref[...], wsb2_ref[...], wv2_ref[...],
                                   False, False)
        m16 = _pack16_t(s3, v3x, v3y, v3z, True)
        y = jnp.concatenate(
            [m16[:, sub * j:sub * j + sub] for j in range(8)], axis=0)
        out_ref[...] = _t(y)

    nblk = e // be
    wsd = jnp.concatenate([m0['ws_w'][0:4].T, m0['ws_w'][36:40].T], axis=1)
    g128 = gout.reshape(-1, 128)
    return pl.pallas_call(
        body,
        grid=(nblk,),
        in_specs=[_rows(be // 8, 128), _rows(be // 8, 128, block_off=nblk),
                  _cols(8, be), _cols(3, be),
                  _full((5, 5)), _full((4, 8)), _full((4, 5)), _full((2, 5)),
                  _full((2, 2)), _full((4, 6)), _full((4, 1)), _full((2, 2)),
                  _full((2, 2)), _full((4, 6)), _full((4, 1)), _full((2, 2))],
        out_specs=_rows(be // 8, 128),
        out_shape=jax.ShapeDtypeStruct((e // 8, 128), F32),
    )(g128, g128, eproj8, evemb,
      m0['wh'].T, wsd, m0['ws_w'][40:45].T, m0['wv'].T,
      m1['wh'].T, m1['ws_w'].T, _col(m1['ws_b']), m1['wv'].T,
      m2['wh'].T, m2['ws_w'].T, _col(m2['ws_b']), m2['wv'].T)


def _update_common(t_ref, p0_ref, p1_ref, lp_refs):
    (g0_ref, b0_ref, fwh0_ref, fwsw0_ref, fwsb0_ref, fwv0_ref,
     fwh1_ref, fwsw1_ref, fwsb1_ref, fwv1_ref, g1_ref, b1_ref) = lp_refs
    t = _unpack128(t_ref[...])
    agg = _unpack128(p0_ref[0]) + _unpack128(p1_ref[0])
    inv = 1.0 / jnp.maximum(agg[10:11], 1.0)
    s = t[0:4] + agg[0:4] * inv
    vx = t[4:6] + agg[4:6] * inv
    vy = t[6:8] + agg[6:8] * inv
    vz = t[8:10] + agg[8:10] * inv
    s = _ln_s_t(s, g0_ref[...], b0_ref[...])
    vx, vy, vz = _ln_v_t(vx, vy, vz)
    hs, hx, hy, hz = _gvp_t(s, vx, vy, vz, fwh0_ref[...], fwsw0_ref[...],
                            fwsb0_ref[...], fwv0_ref[...], True, True)
    hs, hx, hy, hz = _gvp_t(hs, hx, hy, hz, fwh1_ref[...], fwsw1_ref[...],
                            fwsb1_ref[...], fwv1_ref[...], False, False)
    s = _ln_s_t(s + hs, g1_ref[...], b1_ref[...])
    vx, vy, vz = _ln_v_t(vx + hx, vy + hy, vz + hz)
    return s, vx, vy, vz


_UPD_WSPECS = [
    _full((4, 1)), _full((4, 1)),
    _full((4, 2)), _full((16, 8)), _full((16, 1)), _full((4, 4)),
    _full((4, 4)), _full((4, 20)), _full((4, 1)), _full((2, 4)),
    _full((4, 1)), _full((4, 1)),
]


def _upd_weights(lp):
    f0, f1 = lp['ff']
    return (_col(lp['norm0']['gamma']), _col(lp['norm0']['beta']),
            f0['wh'].T, f0['ws_w'].T, _col(f0['ws_b']), f0['wv'].T,
            f1['wh'].T, f1['ws_w'].T, _col(f1['ws_b']), f1['wv'].T,
            _col(lp['norm1']['gamma']), _col(lp['norm1']['beta']))


def _update_call(table, part, lp):
    n = table.shape[0]
    nblk = 1
    br = n // 8

    def body(t_ref, p0_ref, p1_ref, *rest):
        lp_refs, out_ref = rest[:-1], rest[-1]
        s, vx, vy, vz = _update_common(t_ref, p0_ref, p1_ref, lp_refs)
        out_ref[...] = _pack128(_pack16_t(s, vx, vy, vz, False))

    t128 = table.reshape(n // 8, 128)
    p128 = part.reshape(2, n // 8, 128)
    half = lambda h: pl.BlockSpec((1, n // 8, 128), lambda i, h=h: (h, 0, 0))
    return pl.pallas_call(
        body,
        grid=(nblk,),
        in_specs=[_rows(br, 128), half(0), half(1)]
        + _UPD_WSPECS,
        out_specs=_rows(br, 128),
        out_shape=jax.ShapeDtypeStruct((n // 8, 128), F32),
    )(t128, p128, p128, *_upd_weights(lp)).reshape(n, 16)


def _update_head_call(table, part, lp, params):
    n = table.shape[0]
    nblk = 1
    br = n // 8
    og = params['Wout_gvp']

    def body(t_ref, p0_ref, p1_ref, *rest):
        lp_refs = rest[:12]
        (og_ref, ob_ref, owh_ref, owsw_ref, owsb_ref, fw_ref, fb_ref,
         out_ref) = rest[12:]
        s, vx, vy, vz = _update_common(t_ref, p0_ref, p1_ref, lp_refs)
        s = _ln_s_t(s, og_ref[...], ob_ref[...])
        vx, vy, vz = _ln_v_t(vx, vy, vz)
        so, _, _, _ = _gvp_t(s, vx, vy, vz, owh_ref[...], owsw_ref[...],
                             owsb_ref[...], None, True, False)
        o1 = _dot(fw_ref[...], so) + fb_ref[...]
        y = jnp.concatenate(
            [o1[:, br * j:br * j + br] for j in range(8)], axis=0)
        out_ref[...] = _t(y)

    t128 = table.reshape(n // 8, 128)
    p128 = part.reshape(2, n // 8, 128)
    half = lambda h: pl.BlockSpec((1, n // 8, 128), lambda i, h=h: (h, 0, 0))
    return pl.pallas_call(
        body,
        grid=(nblk,),
        in_specs=[_rows(br, 128), half(0), half(1)]
        + _UPD_WSPECS
        + [_full((4, 1)), _full((4, 1)), _full((2, 2)), _full((4, 6)),
           _full((4, 1)), _full((1, 4)), _full((1, 1))],
        out_specs=_rows(br, 8),
        out_shape=jax.ShapeDtypeStruct((n // 8, 8), F32),
    )(t128, p128, p128, *_upd_weights(lp),
      _col(params['Wout_ln']['gamma']), _col(params['Wout_ln']['beta']),
      og['wh'].T, og['ws_w'].T, _col(og['ws_b']),
      params['final_w'].T, params['final_b'].reshape(1, 1))


def _gather_call(table, gidx):
    n = table.shape[0]
    r_total = gidx.shape[0]
    rows_total = r_total * _IW
    rpt = r_total // _NT
    n_chunks = rpt // _CH
    n_stage = n // 1000
    mesh = plsc.VectorSubcoreMesh(core_axis_name="c", subcore_axis_name="s")

    @functools.partial(
        pl.kernel,
        out_type=jax.ShapeDtypeStruct((rows_total, 16), F32),
        mesh=mesh,
        compiler_params=pltpu.CompilerParams(use_tc_tiling_on_sc=False),
        scratch_types=[
            pltpu.VMEM((_CH, _IW), jnp.int32),
            pltpu.VMEM((_CH, _IW), jnp.int32),
            pltpu.VMEM((_CH * _IW, 16), F32),
            pltpu.VMEM((_CH * _IW, 16), F32),
            pltpu.VMEM_SHARED((n, 16), F32),
            pltpu.SemaphoreType.DMA,
            pltpu.SemaphoreType.DMA,
            pltpu.SemaphoreType.DMA,
            pltpu.SemaphoreType.DMA,
        ],
    )
    def k(table_hbm, idx_hbm, out_hbm, idx_a, idx_b, rows_a, rows_b, tab_sh,
          gsem, osem, ia_sem, ib_sem):
        cid = lax.axis_index("c")
        sid = lax.axis_index("s")
        for z in range(-(-n_stage // _NS)):
            c = sid + z * _NS

            @pl.when(c < n_stage)
            def _():
                pltpu.sync_copy(table_hbm.at[pl.ds(c * 1000, 1000)],
                                tab_sh.at[pl.ds(c * 1000, 1000)])
        plsc.subcore_barrier()
        wid = sid * _NC + cid
        row0 = wid * rpt

        def load_idx(ci, ib, isem):
            pltpu.async_copy(
                idx_hbm.at[pl.ds(row0 + ci * _CH, _CH)], ib, isem)

        def wait_idx(ci, ib, isem):
            pltpu.make_async_copy(
                idx_hbm.at[pl.ds(row0 + ci * _CH, _CH)], ib, isem).wait()

        def do_chunk(ci, buf, ib):
            r0 = row0 + ci * _CH
            cps = [pltpu.async_copy(tab_sh.at[ib.at[j]],
                                    buf.at[pl.ds(j * _IW, _IW)], gsem)
                   for j in range(_CH)]
            for c in cps:
                c.wait()
            pltpu.async_copy(buf, out_hbm.at[pl.ds(r0 * _IW, _CH * _IW)],
                             osem)

        def drain(ci, buf):
            r0 = row0 + ci * _CH
            pltpu.make_async_copy(
                buf, out_hbm.at[pl.ds(r0 * _IW, _CH * _IW)], osem).wait()

        load_idx(0, idx_a, ia_sem)

        def pair(p, carry):
            c0 = p * 2
            wait_idx(c0, idx_a, ia_sem)
            load_idx(c0 + 1, idx_b, ib_sem)

            @pl.when(p > 0)
            def _():
                drain(c0 - 2, rows_a)
            do_chunk(c0, rows_a, idx_a)
            wait_idx(c0 + 1, idx_b, ib_sem)

            @pl.when(c0 + 2 < n_chunks)
            def _():
                load_idx(c0 + 2, idx_a, ia_sem)

            @pl.when(p > 0)
            def _():
                drain(c0 - 1, rows_b)
            do_chunk(c0 + 1, rows_b, idx_b)
            return carry

        lax.fori_loop(0, n_chunks // 2, pair, 0)
        drain(n_chunks - 2, rows_a)
        drain(n_chunks - 1, rows_b)

    return k(table, gidx)


def _scatter_call(msgs, didx, zrows, n):
    r_total = didx.shape[0]
    rpt = r_total // _NT
    n_chunks = rpt // _CH
    zch = zrows.shape[0]
    n_z = n // zch
    mesh = plsc.VectorSubcoreMesh(core_axis_name="c", subcore_axis_name="s")

    @functools.partial(
        pl.kernel,
        out_type=jax.ShapeDtypeStruct((2 * n, 16), F32),
        mesh=mesh,
        compiler_params=pltpu.CompilerParams(use_tc_tiling_on_sc=False),
        scratch_types=[
            pltpu.VMEM((_CH, _IW), jnp.int32),
            pltpu.VMEM((_CH, _IW), jnp.int32),
            pltpu.VMEM((_CH * _IW, 16), F32),
            pltpu.VMEM((_CH * _IW, 16), F32),
            pltpu.VMEM_SHARED((n, 16), F32),
            pltpu.SemaphoreType.DMA,
            pltpu.SemaphoreType.DMA,
            pltpu.SemaphoreType.DMA,
        ],
    )
    def k(msg_hbm, idx_hbm, z_hbm, out_hbm, idx_a, idx_b, msg_a, msg_b,
          agg_sh, la_sem, lb_sem, ssem):
        cid = lax.axis_index("c")
        sid = lax.axis_index("s")
        for z in range(-(-n_z // _NS)):
            c = sid + z * _NS

            @pl.when(c < n_z)
            def _():
                pltpu.sync_copy(z_hbm, agg_sh.at[pl.ds(c * zch, zch)])
        plsc.subcore_barrier()
        base = (cid * _NS + sid) * rpt

        def load(ci, ib, mb, lsem):
            r0 = base + ci * _CH
            pltpu.async_copy(idx_hbm.at[pl.ds(r0, _CH)], ib, lsem)
            pltpu.async_copy(
                msg_hbm.at[pl.ds(r0 * _IW, _CH * _IW)], mb, lsem)

        def wait_load(ci, ib, mb, lsem):
            r0 = base + ci * _CH
            pltpu.make_async_copy(
                idx_hbm.at[pl.ds(r0, _CH)], ib, lsem).wait()
            pltpu.make_async_copy(
                msg_hbm.at[pl.ds(r0 * _IW, _CH * _IW)], mb, lsem).wait()

        def scat(ib, mb):
            cps = [pltpu.async_copy(mb.at[pl.ds(j * _IW, _IW)],
                                    agg_sh.at[ib.at[j]], ssem, add=True)
                   for j in range(_CH)]
            for c in cps:
                c.wait()

        load(0, idx_a, msg_a, la_sem)

        def pair(p, carry):
            c0 = p * 2
            wait_load(c0, idx_a, msg_a, la_sem)

            @pl.when(c0 + 1 < n_chunks)
            def _():
                load(c0 + 1, idx_b, msg_b, lb_sem)
            scat(idx_a, msg_a)

            @pl.when(c0 + 1 < n_chunks)
            def _():
                wait_load(c0 + 1, idx_b, msg_b, lb_sem)

                @pl.when(c0 + 2 < n_chunks)
                def _():
                    load(c0 + 2, idx_a, msg_a, la_sem)
                scat(idx_b, msg_b)
            return carry

        lax.fori_loop(0, (n_chunks + 1) // 2, pair, 0)
        plsc.subcore_barrier()
        for z in range(-(-n_z // _NS)):
            c = sid + z * _NS

            @pl.when(c < n_z)
            def _():
                pltpu.sync_copy(agg_sh.at[pl.ds(c * zch, zch)],
                                out_hbm.at[pl.ds(cid * n + c * zch, zch)])

    return k(msgs, didx, zrows)


def kernel(node_s, node_v, edge_index, edge_s, edge_v, params):
    n = node_s.shape[0]
    e = edge_s.shape[0]
    ns_t = node_s.T
    nv_t = jnp.transpose(node_v, (2, 1, 0)).reshape(6, n)
    es_t = edge_s.T
    ev_t = jnp.transpose(edge_v, (2, 1, 0)).reshape(3, e)
    def _perm(a):
        return a.reshape(-1, 8, 2000).swapaxes(1, 2).reshape(a.shape)

    gidx = _perm(edge_index.reshape(-1)).reshape(2 * e // _IW, _IW)
    didx = gidx[e // _IW:]
    zrows = jnp.zeros((1000, 16), F32)

    table = _node_embed_call(ns_t, nv_t, params)
    eproj8, evemb = _edge_embed_call(es_t, ev_t, params)

    out = None
    for li, lp in enumerate(params['layers']):
        gout = _gather_call(table, gidx)
        msgs = _msg_call(gout, eproj8, evemb, lp, li)
        part = _scatter_call(msgs.reshape(e, 16), didx, zrows, n)
        if li == 0:
            table = _update_call(table, part, lp)
        else:
            out = _update_head_call(table, part, lp, params)
    return out.reshape(n)
